# baseline (device time: 70036 ns/iter reference)
import sys

import jax
import jax.numpy as jnp
from jax import lax
from jax.experimental import pallas as pl
from jax.experimental.pallas import tpu as pltpu

N_DEV = 16
P = 4
M = 1024
D = 1024
HC = D // 2
QR = M // P
CR = QR // P

MESH = pl.DeviceIdType.MESH

try:
    _devs = jax.devices()
    print(f"[topology probe] n={len(_devs)}", file=sys.stderr)
except Exception as _e:
    print(f"[topology probe] failed: {_e}", file=sys.stderr)


def kernel(partial, resid, gamma):
    gamma2 = gamma.reshape(1, D)
    x2 = partial.reshape(M, D)

    def body(x_ref, resid_ref, gamma_ref, out_ref,
             prs, prs2, zrs, send_sems, recv_sems):
        my = lax.axis_index("i")
        g = lax.div(my, P)
        q = lax.rem(my, P)

        p_right = g * P + lax.rem(q + 1, P)
        p_left = g * P + lax.rem(q + 3, P)
        z_up = lax.rem(g + 1, P) * P + q
        z_dn = lax.rem(g + 3, P) * P + q

        p_diag = g * P + lax.rem(q + 2, P)
        z_2 = lax.rem(g + 2, P) * P + q

        barrier = pltpu.get_barrier_semaphore()
        for nbr in (p_right, p_left, z_up, z_dn, p_diag, z_2):
            pl.semaphore_signal(barrier, inc=1, device_id=(nbr,),
                                device_id_type=MESH)
        pl.semaphore_wait(barrier, 6)

        cols = (pl.ds(0, HC), pl.ds(HC, HC))
        all_rdmas = []
        qb = lax.rem(q + 1, P) * QR

        def remote(src, dst, slot, tgt):
            rdma = pltpu.make_async_remote_copy(
                src_ref=src, dst_ref=dst,
                send_sem=send_sems.at[slot],
                recv_sem=recv_sems.at[slot],
                device_id=(tgt,), device_id_type=MESH)
            rdma.start()
            all_rdmas.append(rdma)
            return rdma

        for h in range(2):
            hop = []
            for d in range(2):
                if d == 0:
                    sc = lax.rem(q - h + P, P)
                    rc = lax.rem(q - h + P - 1, P)
                    tgt = p_right
                else:
                    sc = lax.rem(q + h + 2, P)
                    rc = lax.rem(q + h + 3, P)
                    tgt = p_left
                slot = d * 2 + h
                src = x_ref if h == 0 else out_ref
                rdma = remote(src.at[pl.ds(sc * QR, QR), cols[d]],
                              prs.at[slot], slot, tgt)
                hop.append((rdma, rc, slot, d))
            for rdma, rc, slot, d in hop:
                rdma.wait_recv()
                out_ref[pl.ds(rc * QR, QR), cols[d]] = (
                    x_ref[pl.ds(rc * QR, QR), cols[d]] + prs[slot])

        def b_rs(d, w):
            if d == 0:
                return lax.rem(g - w + P, P)
            return lax.rem(g + 2 + w, P)

        def p2_start(d, w):
            b = b_rs(d, w)
            if d == 0:
                a = lax.rem(q - 2 + P, P)
                tgt = p_right
            else:
                a = q
                tgt = p_left
            slot = 4 + d * P + w
            return remote(out_ref.at[pl.ds(a * QR + b * CR, CR), cols[d]],
                          prs2.at[d * P + w], slot, tgt)

        def p2_fin(d, w, rdma):
            rdma.wait_recv()
            b = b_rs(d, w)
            rows = pl.ds(qb + b * CR, CR)
            out_ref[rows, cols[d]] = x_ref[rows, cols[d]] + prs2[d * P + w]

        ob = qb + lax.rem(g + 1, P) * CR

        def tr_send(d, kind):
            w = (0, 2, 1)[kind]
            b = b_rs(d, w)
            if kind == 2:
                tgt = z_2
            else:
                tgt = z_dn if d == 0 else z_up
            slot = 12 + d * 3 + kind
            return remote(out_ref.at[pl.ds(qb + b * CR, CR), cols[d]],
                          zrs.at[d * 3 + kind], slot, tgt)

        pend = {}
        for d in (0, 1):
            pend[("p2", d, 0)] = p2_start(d, 0)
        for d in (0, 1):
            p2_fin(d, 0, pend.pop(("p2", d, 0)))
            pend[("tr", d, 0)] = tr_send(d, 0)
            pend[("p2", d, 1)] = p2_start(d, 1)
        for d in (0, 1):
            p2_fin(d, 1, pend.pop(("p2", d, 1)))
            pend[("p2", d, 2)] = p2_start(d, 2)
        for d in (0, 1):
            p2_fin(d, 2, pend.pop(("p2", d, 2)))
            pend[("tr", d, 1)] = tr_send(d, 1)
            pend[("p2", d, 3)] = p2_start(d, 3)
        for d in (0, 1):
            p2_fin(d, 3, pend.pop(("p2", d, 3)))
        for d in (0, 1):
            pend.pop(("tr", d, 1)).wait_recv()
            out_ref[pl.ds(qb + b_rs(d, 1) * CR, CR), cols[d]] += (
                zrs[d * 3 + 1])
            pend[("tr", d, 2)] = tr_send(d, 2)
        for d in (0, 1):
            pend.pop(("tr", d, 0)).wait_recv()
            out_ref[pl.ds(ob, CR), cols[d]] += zrs[d * 3 + 0]
        for d in (0, 1):
            pend.pop(("tr", d, 2)).wait_recv()
            out_ref[pl.ds(ob, CR), cols[d]] += zrs[d * 3 + 2]
        assert not pend

        y = out_ref[pl.ds(ob, CR), :] + resid_ref[pl.ds(ob, CR), :]
        rms = jnp.sqrt(jnp.mean(y * y, axis=-1, keepdims=True) + 1e-6)
        out_ref[pl.ds(ob, CR), :] = y / rms * gamma_ref[:, :]

        def zag_start(d, h):
            if d == 0:
                sc = lax.rem(g + 1 - h + P, P)
                tgt = z_up
            else:
                sc = lax.rem(g + 1 + h, P)
                tgt = z_dn
            slot = 18 + d * 3 + h
            chunk = out_ref.at[pl.ds(qb + sc * CR, CR), cols[d]]
            pend[("zag", d, h)] = remote(chunk, chunk, slot, tgt)

        def zag_wait(d, h):
            pend.pop(("zag", d, h)).wait_recv()

        def wave_b(d, w):
            if w == 0:
                return lax.rem(g + 1, P)
            if d == 0:
                return lax.rem(g + 1 - w + P, P)
            return lax.rem(g + 1 + w, P)

        def wave_start(d, w, h):
            b = wave_b(d, w)
            if d == 0:
                a = lax.rem(q + 1 - h + P, P)
                tgt = p_right
            else:
                a = lax.rem(q + 1 + h, P)
                tgt = p_left
            slot = 24 + d * 12 + w * 3 + h
            chunk = out_ref.at[pl.ds(a * QR + b * CR, CR), cols[d]]
            pend[("w", d, w, h)] = remote(chunk, chunk, slot, tgt)

        def wave_wait(d, w, h):
            pend.pop(("w", d, w, h)).wait_recv()

        for d in (0, 1):
            zag_start(d, 0)
            wave_start(d, 0, 0)
        for d in (0, 1):
            wave_wait(d, 0, 0)
            wave_start(d, 0, 1)
        for d in (0, 1):
            zag_wait(d, 0)
            zag_start(d, 1)
            wave_start(d, 1, 0)
        for d in (0, 1):
            wave_wait(d, 0, 1)
            wave_start(d, 0, 2)
        for d in (0, 1):
            wave_wait(d, 1, 0)
            wave_start(d, 1, 1)
        for d in (0, 1):
            zag_wait(d, 1)
            zag_start(d, 2)
            wave_start(d, 2, 0)
        for d in (0, 1):
            wave_wait(d, 1, 1)
            wave_start(d, 1, 2)
        for d in (0, 1):
            wave_wait(d, 2, 0)
            wave_start(d, 2, 1)
        def w3d_send(d, j):
            b = wave_b(d, 3)
            if d == 0:
                tgt = g * P + lax.rem(q + 1 + j, P)
            else:
                tgt = g * P + lax.rem(q - 1 - j + 2 * P, P)
            slot = 24 + d * 12 + 9 + j
            chunk = out_ref.at[pl.ds(qb + b * CR, CR), cols[d]]
            pend[("w3d", d, j)] = remote(chunk, chunk, slot, tgt)

        def w3d_wait(d, j):
            pend.pop(("w3d", d, j)).wait_recv()

        for d in (0, 1):
            zag_wait(d, 2)
            for j in range(3):
                w3d_send(d, j)
        for d in (0, 1):
            wave_wait(d, 2, 1)
            wave_start(d, 2, 2)
        for d in (0, 1):
            for j in range(3):
                w3d_wait(d, j)
        for d in (0, 1):
            for w in range(3):
                wave_wait(d, w, 2)
        assert not pend

        for rdma in all_rdmas:
            rdma.wait_send()

    return pl.pallas_call(
        body,
        out_shape=jax.ShapeDtypeStruct((M, D), jnp.float32),
        in_specs=[
            pl.BlockSpec(memory_space=pltpu.VMEM),
            pl.BlockSpec(memory_space=pltpu.VMEM),
            pl.BlockSpec(memory_space=pltpu.VMEM),
        ],
        out_specs=pl.BlockSpec(memory_space=pltpu.VMEM),
        scratch_shapes=[
            pltpu.VMEM((4, QR, HC), jnp.float32),
            pltpu.VMEM((8, CR, HC), jnp.float32),
            pltpu.VMEM((6, CR, HC), jnp.float32),
            pltpu.SemaphoreType.DMA((48,)),
            pltpu.SemaphoreType.DMA((48,)),
        ],
        compiler_params=pltpu.CompilerParams(collective_id=0),
    )(x2, resid, gamma2)


# device time: 65538 ns/iter; 1.0686x vs baseline; 1.0686x over previous
import sys

import jax
import jax.numpy as jnp
from jax import lax
from jax.experimental import pallas as pl
from jax.experimental.pallas import tpu as pltpu

N_DEV = 16
P = 4
M = 1024
D = 1024
HC = D // 2
QR = M // P
CR = QR // P

MESH = pl.DeviceIdType.MESH

try:
    _devs = jax.devices()
    print(f"[topology probe] n={len(_devs)}", file=sys.stderr)
except Exception as _e:
    print(f"[topology probe] failed: {_e}", file=sys.stderr)


def kernel(partial, resid, gamma):
    gamma2 = gamma.reshape(1, D)
    x2 = partial.reshape(M, D)

    def body(x_ref, resid_ref, gamma_ref, out_ref,
             prs, prs2, zrs, send_sems, recv_sems):
        my = lax.axis_index("i")
        g = lax.div(my, P)
        q = lax.rem(my, P)

        p_right = g * P + lax.rem(q + 1, P)
        p_left = g * P + lax.rem(q + 3, P)
        z_up = lax.rem(g + 1, P) * P + q
        z_dn = lax.rem(g + 3, P) * P + q

        p_diag = g * P + lax.rem(q + 2, P)

        barrier = pltpu.get_barrier_semaphore()
        for nbr in (p_right, p_left, z_up, z_dn, p_diag):
            pl.semaphore_signal(barrier, inc=1, device_id=(nbr,),
                                device_id_type=MESH)
        pl.semaphore_wait(barrier, 5)

        cols = (pl.ds(0, HC), pl.ds(HC, HC))
        all_rdmas = []
        qb = lax.rem(q + 1, P) * QR

        def remote(src, dst, slot, tgt):
            rdma = pltpu.make_async_remote_copy(
                src_ref=src, dst_ref=dst,
                send_sem=send_sems.at[slot],
                recv_sem=recv_sems.at[slot],
                device_id=(tgt,), device_id_type=MESH)
            rdma.start()
            all_rdmas.append(rdma)
            return rdma

        for h in range(2):
            hop = []
            for d in range(2):
                if d == 0:
                    sc = lax.rem(q - h + P, P)
                    rc = lax.rem(q - h + P - 1, P)
                    tgt = p_right
                else:
                    sc = lax.rem(q + h + 2, P)
                    rc = lax.rem(q + h + 3, P)
                    tgt = p_left
                slot = d * 2 + h
                src = x_ref if h == 0 else out_ref
                rdma = remote(src.at[pl.ds(sc * QR, QR), cols[d]],
                              prs.at[slot], slot, tgt)
                hop.append((rdma, rc, slot, d))
            for rdma, rc, slot, d in hop:
                rdma.wait_recv()
                out_ref[pl.ds(rc * QR, QR), cols[d]] = (
                    x_ref[pl.ds(rc * QR, QR), cols[d]] + prs[slot])

        def b_rs(d, w):
            if d == 0:
                return lax.rem(g - w + P, P)
            return lax.rem(g + 2 + w, P)

        def p2_start(d, w):
            b = b_rs(d, w)
            if d == 0:
                a = lax.rem(q - 2 + P, P)
                tgt = p_right
            else:
                a = q
                tgt = p_left
            slot = 4 + d * P + w
            return remote(out_ref.at[pl.ds(a * QR + b * CR, CR), cols[d]],
                          prs2.at[d * P + w], slot, tgt)

        def p2_fin(d, w, rdma):
            rdma.wait_recv()
            b = b_rs(d, w)
            rows = pl.ds(qb + b * CR, CR)
            out_ref[rows, cols[d]] = x_ref[rows, cols[d]] + prs2[d * P + w]

        def z_start(d, h):
            if d == 0:
                sc = lax.rem(g - h + P, P)
                tgt = z_up
            else:
                sc = lax.rem(g + h + 2, P)
                tgt = z_dn
            slot = 12 + d * 3 + h
            return remote(out_ref.at[pl.ds(qb + sc * CR, CR), cols[d]],
                          zrs.at[d * 3 + h], slot, tgt)

        def z_fin(d, h, rdma):
            rdma.wait_recv()
            rc = lax.rem(g - h + P - 1, P) if d == 0 else lax.rem(g + h + 3, P)
            out_ref[pl.ds(qb + rc * CR, CR), cols[d]] += zrs[d * 3 + h]

        pend = {}
        for d in (0, 1):
            pend[("p2", d, 0)] = p2_start(d, 0)
        for d in (0, 1):
            p2_fin(d, 0, pend.pop(("p2", d, 0)))
            pend[("z", d, 0)] = z_start(d, 0)
            pend[("p2", d, 1)] = p2_start(d, 1)
        for d in (0, 1):
            p2_fin(d, 1, pend.pop(("p2", d, 1)))
            pend[("p2", d, 2)] = p2_start(d, 2)
        for d in (0, 1):
            z_fin(d, 0, pend.pop(("z", d, 0)))
            pend[("z", d, 1)] = z_start(d, 1)
        for d in (0, 1):
            p2_fin(d, 2, pend.pop(("p2", d, 2)))
            pend[("p2", d, 3)] = p2_start(d, 3)
        for d in (0, 1):
            z_fin(d, 1, pend.pop(("z", d, 1)))
            pend[("z", d, 2)] = z_start(d, 2)
        for d in (0, 1):
            p2_fin(d, 3, pend.pop(("p2", d, 3)))
        for d in (0, 1):
            z_fin(d, 2, pend.pop(("z", d, 2)))
        assert not pend

        ob = qb + lax.rem(g + 1, P) * CR
        y = out_ref[pl.ds(ob, CR), :] + resid_ref[pl.ds(ob, CR), :]
        rms = jnp.sqrt(jnp.mean(y * y, axis=-1, keepdims=True) + 1e-6)
        out_ref[pl.ds(ob, CR), :] = y / rms * gamma_ref[:, :]

        def zag_start(d, h):
            if d == 0:
                sc = lax.rem(g + 1 - h + P, P)
                tgt = z_up
            else:
                sc = lax.rem(g + 1 + h, P)
                tgt = z_dn
            slot = 18 + d * 3 + h
            chunk = out_ref.at[pl.ds(qb + sc * CR, CR), cols[d]]
            pend[("zag", d, h)] = remote(chunk, chunk, slot, tgt)

        def zag_wait(d, h):
            pend.pop(("zag", d, h)).wait_recv()

        def wave_b(d, w):
            if w == 0:
                return lax.rem(g + 1, P)
            if d == 0:
                return lax.rem(g + 1 - w + P, P)
            return lax.rem(g + 1 + w, P)

        def wave_start(d, w, h):
            b = wave_b(d, w)
            if d == 0:
                a = lax.rem(q + 1 - h + P, P)
                tgt = p_right
            else:
                a = lax.rem(q + 1 + h, P)
                tgt = p_left
            slot = 24 + d * 12 + w * 3 + h
            chunk = out_ref.at[pl.ds(a * QR + b * CR, CR), cols[d]]
            pend[("w", d, w, h)] = remote(chunk, chunk, slot, tgt)

        def wave_wait(d, w, h):
            pend.pop(("w", d, w, h)).wait_recv()

        for d in (0, 1):
            zag_start(d, 0)
            wave_start(d, 0, 0)
        for d in (0, 1):
            wave_wait(d, 0, 0)
            wave_start(d, 0, 1)
        for d in (0, 1):
            zag_wait(d, 0)
            zag_start(d, 1)
            wave_start(d, 1, 0)
        for d in (0, 1):
            wave_wait(d, 0, 1)
            wave_start(d, 0, 2)
        for d in (0, 1):
            wave_wait(d, 1, 0)
            wave_start(d, 1, 1)
        for d in (0, 1):
            zag_wait(d, 1)
            zag_start(d, 2)
            wave_start(d, 2, 0)
        for d in (0, 1):
            wave_wait(d, 1, 1)
            wave_start(d, 1, 2)
        for d in (0, 1):
            wave_wait(d, 2, 0)
            wave_start(d, 2, 1)
        def w3d_send(d, j):
            b = wave_b(d, 3)
            if d == 0:
                tgt = g * P + lax.rem(q + 1 + j, P)
            else:
                tgt = g * P + lax.rem(q - 1 - j + 2 * P, P)
            slot = 24 + d * 12 + 9 + j
            chunk = out_ref.at[pl.ds(qb + b * CR, CR), cols[d]]
            pend[("w3d", d, j)] = remote(chunk, chunk, slot, tgt)

        def w3d_wait(d, j):
            pend.pop(("w3d", d, j)).wait_recv()

        for d in (0, 1):
            zag_wait(d, 2)
            for j in range(3):
                w3d_send(d, j)
        for d in (0, 1):
            wave_wait(d, 2, 1)
            wave_start(d, 2, 2)
        for d in (0, 1):
            for j in range(3):
                w3d_wait(d, j)
        for d in (0, 1):
            for w in range(3):
                wave_wait(d, w, 2)
        assert not pend

        for rdma in all_rdmas:
            rdma.wait_send()

    return pl.pallas_call(
        body,
        out_shape=jax.ShapeDtypeStruct((M, D), jnp.float32),
        in_specs=[
            pl.BlockSpec(memory_space=pltpu.VMEM),
            pl.BlockSpec(memory_space=pltpu.VMEM),
            pl.BlockSpec(memory_space=pltpu.VMEM),
        ],
        out_specs=pl.BlockSpec(memory_space=pltpu.VMEM),
        scratch_shapes=[
            pltpu.VMEM((4, QR, HC), jnp.float32),
            pltpu.VMEM((8, CR, HC), jnp.float32),
            pltpu.VMEM((6, CR, HC), jnp.float32),
            pltpu.SemaphoreType.DMA((48,)),
            pltpu.SemaphoreType.DMA((48,)),
        ],
        compiler_params=pltpu.CompilerParams(collective_id=0),
    )(x2, resid, gamma2)


# device time: 65397 ns/iter; 1.0709x vs baseline; 1.0022x over previous
import sys

import jax
import jax.numpy as jnp
from jax import lax
from jax.experimental import pallas as pl
from jax.experimental.pallas import tpu as pltpu

N_DEV = 16
P = 4
M = 1024
D = 1024
HC = D // 2
QR = M // P
CR = QR // P

MESH = pl.DeviceIdType.MESH

try:
    _devs = jax.devices()
    print(f"[topology probe] n={len(_devs)}", file=sys.stderr)
except Exception as _e:
    print(f"[topology probe] failed: {_e}", file=sys.stderr)


def kernel(partial, resid, gamma):
    gamma2 = gamma.reshape(1, D)
    x2 = partial.reshape(M, D)

    def body(x_ref, resid_ref, gamma_ref, out_ref,
             prs, prs2, zrs, send_sems, recv_sems):
        my = lax.axis_index("i")
        g = lax.div(my, P)
        q = lax.rem(my, P)

        p_right = g * P + lax.rem(q + 1, P)
        p_left = g * P + lax.rem(q + 3, P)
        z_up = lax.rem(g + 1, P) * P + q
        z_dn = lax.rem(g + 3, P) * P + q

        p_diag = g * P + lax.rem(q + 2, P)

        barrier = pltpu.get_barrier_semaphore()
        for nbr in (p_right, p_left, z_up, z_dn, p_diag):
            pl.semaphore_signal(barrier, inc=1, device_id=(nbr,),
                                device_id_type=MESH)
        pl.semaphore_wait(barrier, 5)

        cols = (pl.ds(0, HC), pl.ds(HC, HC))
        all_rdmas = []
        qb = lax.rem(q + 1, P) * QR

        def remote(src, dst, slot, tgt):
            rdma = pltpu.make_async_remote_copy(
                src_ref=src, dst_ref=dst,
                send_sem=send_sems.at[slot],
                recv_sem=recv_sems.at[slot],
                device_id=(tgt,), device_id_type=MESH)
            rdma.start()
            all_rdmas.append(rdma)
            return rdma

        def p1_start(d, h):
            if d == 0:
                sc = lax.rem(q - h + P, P)
                tgt = p_right
            else:
                sc = lax.rem(q + h + 2, P)
                tgt = p_left
            slot = d * 2 + h
            src = x_ref if h == 0 else out_ref
            return remote(src.at[pl.ds(sc * QR, QR), cols[d]],
                          prs.at[slot], slot, tgt)

        def p1_fin(d, h, rdma):
            rdma.wait_recv()
            rc = lax.rem(q - h + P - 1, P) if d == 0 else lax.rem(q + h + 3, P)
            out_ref[pl.ds(rc * QR, QR), cols[d]] = (
                x_ref[pl.ds(rc * QR, QR), cols[d]] + prs[d * 2 + h])

        h0 = [p1_start(d, 0) for d in (0, 1)]
        h1 = []
        for d in (0, 1):
            p1_fin(d, 0, h0[d])
            h1.append(p1_start(d, 1))

        def b_rs(d, w):
            if d == 0:
                return lax.rem(g - w + P, P)
            return lax.rem(g + 2 + w, P)

        def p2_start(d, w):
            b = b_rs(d, w)
            if d == 0:
                a = lax.rem(q - 2 + P, P)
                tgt = p_right
            else:
                a = q
                tgt = p_left
            slot = 4 + d * P + w
            return remote(out_ref.at[pl.ds(a * QR + b * CR, CR), cols[d]],
                          prs2.at[d * P + w], slot, tgt)

        def p2_fin(d, w, rdma):
            rdma.wait_recv()
            b = b_rs(d, w)
            rows = pl.ds(qb + b * CR, CR)
            out_ref[rows, cols[d]] = x_ref[rows, cols[d]] + prs2[d * P + w]

        def z_start(d, h):
            if d == 0:
                sc = lax.rem(g - h + P, P)
                tgt = z_up
            else:
                sc = lax.rem(g + h + 2, P)
                tgt = z_dn
            slot = 12 + d * 3 + h
            return remote(out_ref.at[pl.ds(qb + sc * CR, CR), cols[d]],
                          zrs.at[d * 3 + h], slot, tgt)

        def z_fin(d, h, rdma):
            rdma.wait_recv()
            rc = lax.rem(g - h + P - 1, P) if d == 0 else lax.rem(g + h + 3, P)
            out_ref[pl.ds(qb + rc * CR, CR), cols[d]] += zrs[d * 3 + h]

        pend = {}
        for d in (0, 1):
            p1_fin(d, 1, h1[d])
            pend[("p2", d, 0)] = p2_start(d, 0)
        for d in (0, 1):
            p2_fin(d, 0, pend.pop(("p2", d, 0)))
            pend[("z", d, 0)] = z_start(d, 0)
            pend[("p2", d, 1)] = p2_start(d, 1)
        for d in (0, 1):
            p2_fin(d, 1, pend.pop(("p2", d, 1)))
            pend[("p2", d, 2)] = p2_start(d, 2)
        for d in (0, 1):
            z_fin(d, 0, pend.pop(("z", d, 0)))
            pend[("z", d, 1)] = z_start(d, 1)
        for d in (0, 1):
            p2_fin(d, 2, pend.pop(("p2", d, 2)))
            pend[("p2", d, 3)] = p2_start(d, 3)
        for d in (0, 1):
            z_fin(d, 1, pend.pop(("z", d, 1)))
            pend[("z", d, 2)] = z_start(d, 2)
        for d in (0, 1):
            p2_fin(d, 3, pend.pop(("p2", d, 3)))
        for d in (0, 1):
            z_fin(d, 2, pend.pop(("z", d, 2)))
        assert not pend

        ob = qb + lax.rem(g + 1, P) * CR
        y = out_ref[pl.ds(ob, CR), :] + resid_ref[pl.ds(ob, CR), :]
        rms = jnp.sqrt(jnp.mean(y * y, axis=-1, keepdims=True) + 1e-6)
        out_ref[pl.ds(ob, CR), :] = y / rms * gamma_ref[:, :]

        def zag_start(d, h):
            if d == 0:
                sc = lax.rem(g + 1 - h + P, P)
                tgt = z_up
            else:
                sc = lax.rem(g + 1 + h, P)
                tgt = z_dn
            slot = 18 + d * 3 + h
            chunk = out_ref.at[pl.ds(qb + sc * CR, CR), cols[d]]
            pend[("zag", d, h)] = remote(chunk, chunk, slot, tgt)

        def zag_wait(d, h):
            pend.pop(("zag", d, h)).wait_recv()

        def wave_b(d, w):
            if w == 0:
                return lax.rem(g + 1, P)
            if d == 0:
                return lax.rem(g + 1 - w + P, P)
            return lax.rem(g + 1 + w, P)

        def wave_start(d, w, h):
            b = wave_b(d, w)
            if d == 0:
                a = lax.rem(q + 1 - h + P, P)
                tgt = p_right
            else:
                a = lax.rem(q + 1 + h, P)
                tgt = p_left
            slot = 24 + d * 12 + w * 3 + h
            chunk = out_ref.at[pl.ds(a * QR + b * CR, CR), cols[d]]
            pend[("w", d, w, h)] = remote(chunk, chunk, slot, tgt)

        def wave_wait(d, w, h):
            pend.pop(("w", d, w, h)).wait_recv()

        for d in (0, 1):
            zag_start(d, 0)
            wave_start(d, 0, 0)
        for d in (0, 1):
            wave_wait(d, 0, 0)
            wave_start(d, 0, 1)
        for d in (0, 1):
            zag_wait(d, 0)
            zag_start(d, 1)
            wave_start(d, 1, 0)
        for d in (0, 1):
            wave_wait(d, 0, 1)
            wave_start(d, 0, 2)
        for d in (0, 1):
            wave_wait(d, 1, 0)
            wave_start(d, 1, 1)
        for d in (0, 1):
            zag_wait(d, 1)
            zag_start(d, 2)
            wave_start(d, 2, 0)
        for d in (0, 1):
            wave_wait(d, 1, 1)
            wave_start(d, 1, 2)
        for d in (0, 1):
            wave_wait(d, 2, 0)
            wave_start(d, 2, 1)
        def w3d_send(d, j):
            b = wave_b(d, 3)
            if d == 0:
                tgt = g * P + lax.rem(q + 1 + j, P)
            else:
                tgt = g * P + lax.rem(q - 1 - j + 2 * P, P)
            slot = 24 + d * 12 + 9 + j
            chunk = out_ref.at[pl.ds(qb + b * CR, CR), cols[d]]
            pend[("w3d", d, j)] = remote(chunk, chunk, slot, tgt)

        def w3d_wait(d, j):
            pend.pop(("w3d", d, j)).wait_recv()

        for d in (0, 1):
            zag_wait(d, 2)
            for j in range(3):
                w3d_send(d, j)
        for d in (0, 1):
            wave_wait(d, 2, 1)
            wave_start(d, 2, 2)
        for d in (0, 1):
            for j in range(3):
                w3d_wait(d, j)
        for d in (0, 1):
            for w in range(3):
                wave_wait(d, w, 2)
        assert not pend

        for rdma in all_rdmas:
            rdma.wait_send()

    return pl.pallas_call(
        body,
        out_shape=jax.ShapeDtypeStruct((M, D), jnp.float32),
        in_specs=[
            pl.BlockSpec(memory_space=pltpu.VMEM),
            pl.BlockSpec(memory_space=pltpu.VMEM),
            pl.BlockSpec(memory_space=pltpu.VMEM),
        ],
        out_specs=pl.BlockSpec(memory_space=pltpu.VMEM),
        scratch_shapes=[
            pltpu.VMEM((4, QR, HC), jnp.float32),
            pltpu.VMEM((8, CR, HC), jnp.float32),
            pltpu.VMEM((6, CR, HC), jnp.float32),
            pltpu.SemaphoreType.DMA((48,)),
            pltpu.SemaphoreType.DMA((48,)),
        ],
        compiler_params=pltpu.CompilerParams(collective_id=0),
    )(x2, resid, gamma2)
